# Initial kernel scaffold; baseline (speedup 1.0000x reference)
import jax
import jax.numpy as jnp
from jax.experimental import pallas as pl


def kernel(x, homo_edge_index, hetero_edge_index, W_center1, W_homo1, W_hetero1, W_center2, W_homo2, W_hetero2):
    raise NotImplementedError("write your pallas kernel here")


# speedup vs baseline: 13.1292x; 13.1292x over previous
import functools

import jax
import jax.numpy as jnp
from jax import lax
from jax.experimental import pallas as pl
from jax.experimental.pallas import tpu as pltpu
from jax.experimental.pallas import tpu_sc as plsc

N = 10000
E = 160000
D_IN = 256
D_HID = 256
D_OUT = 128
BETA = 0.5

NC = 2
NS = 16
L = 16

CH = 128
NCHUNK = 1280
NW = 40
EP = NCHUNK * CH
NA = 10240
APT = NA // NS
WB = N // NS
BLK = 1000
GRID = N // BLK

_mesh = plsc.VectorSubcoreMesh(core_axis_name="c", subcore_axis_name="s")


def _fill(ref, n, vec):
    def body(i, _):
        ref[pl.ds(i * L, L)] = vec
        return 0
    lax.fori_loop(0, n // L, body, 0)


@functools.partial(
    pl.kernel,
    out_type=jax.ShapeDtypeStruct((NC, NA), jnp.float32),
    mesh=_mesh,
    scratch_types=[
        pltpu.VMEM((NCHUNK // NS, CH), jnp.int32),
        pltpu.VMEM((CH,), jnp.float32),
        pltpu.VMEM((APT,), jnp.float32),
        pltpu.VMEM_SHARED((NA,), jnp.float32),
    ],
)
def _deg_kernel(colh_hbm, colt_hbm, deg_out, idx_v, ones_v, zeros_v, acc):
    c = lax.axis_index("c")
    t = lax.axis_index("s")
    cpt = NCHUNK // NS
    _fill(ones_v, CH, jnp.ones((L,), jnp.float32))
    _fill(zeros_v, APT, jnp.zeros((L,), jnp.float32))
    pltpu.sync_copy(zeros_v, acc.at[pl.ds(t * APT, APT)])

    @pl.when(c == 0)
    def _():
        pltpu.sync_copy(colh_hbm.at[pl.ds(t * cpt, cpt)], idx_v)

    @pl.when(c == 1)
    def _():
        pltpu.sync_copy(colt_hbm.at[pl.ds(t * cpt, cpt)], idx_v)

    plsc.subcore_barrier()

    def body(j, _):
        pltpu.sync_copy(ones_v, acc.at[idx_v.at[j]], add=True)
        return 0
    lax.fori_loop(0, cpt, body, 0)

    plsc.subcore_barrier()
    pltpu.sync_copy(acc.at[pl.ds(t * APT, APT)],
                    deg_out.at[c, pl.ds(t * APT, APT)])


def _make_conv(split_features):
    cpt = NCHUNK // NS if split_features else NCHUNK // (NC * NS)

    @functools.partial(
        pl.kernel,
        out_type=[jax.ShapeDtypeStruct((NC, NA, 128), jnp.float32)] * 2,
        mesh=_mesh,
        scratch_types=[
            pltpu.VMEM((NW, CH), jnp.int32),
            pltpu.VMEM((NW, CH), jnp.int32),
            pltpu.VMEM((CH, 128), jnp.float32),
            pltpu.VMEM((CH, 128), jnp.float32),
            pltpu.VMEM_SHARED((NA, 128), jnp.float32),
            pltpu.SemaphoreType.DMA,
            pltpu.SemaphoreType.DMA,
        ],
    )
    def conv(gh_hbm, gt_hbm, rowh, colh, rowt, colt, outh, outt,
             idxr, idxc, bufa, bufb, acc, sema, semb):
        c = lax.axis_index("c")
        t = lax.axis_index("s")

        base = t * cpt if split_features else c * (NCHUNK // NC) + t * cpt

        def one_conv(g_hbm, row_hbm, col_hbm, out_hbm):
            def zb(i, _):
                bufa[i // 8, pl.ds((i % 8) * L, L)] = jnp.zeros((L,),
                                                                jnp.float32)
                return 0
            lax.fori_loop(0, CH * 8, zb, 0)
            for z in range(APT // CH):
                pltpu.sync_copy(bufa, acc.at[pl.ds(t * APT + z * CH, CH)])
            plsc.subcore_barrier()

            for w in range(cpt // NW):
                wbase = base + w * NW
                pltpu.sync_copy(row_hbm.at[pl.ds(wbase, NW)], idxr)
                pltpu.sync_copy(col_hbm.at[pl.ds(wbase, NW)], idxc)
                if split_features:
                    def tr(i, _):
                        j = i // 8
                        k = i % 8
                        v = idxr[j, pl.ds(k * L, L)]
                        idxr[j, pl.ds(k * L, L)] = v * 2 + c
                        return 0
                    lax.fori_loop(0, NW * 8, tr, 0)

                pltpu.async_copy(g_hbm.at[idxr.at[0]], bufa, sema)

                def step(jj, _):
                    c0 = 2 * jj
                    c1 = 2 * jj + 1
                    pltpu.make_async_copy(g_hbm.at[idxr.at[0]], bufa,
                                          sema).wait()
                    pltpu.async_copy(g_hbm.at[idxr.at[c1]], bufb, semb)
                    pltpu.sync_copy(bufa, acc.at[idxc.at[c0]], add=True)
                    pltpu.make_async_copy(g_hbm.at[idxr.at[0]], bufb,
                                          semb).wait()
                    nxt = jnp.minimum(c0 + 2, NW - 1)
                    pltpu.async_copy(g_hbm.at[idxr.at[nxt]], bufa, sema)
                    pltpu.sync_copy(bufb, acc.at[idxc.at[c1]], add=True)
                    return 0
                lax.fori_loop(0, NW // 2, step, 0)
                pltpu.make_async_copy(g_hbm.at[idxr.at[0]], bufa, sema).wait()

            plsc.subcore_barrier()
            pltpu.sync_copy(acc.at[pl.ds(t * APT, APT)],
                            out_hbm.at[c, pl.ds(t * APT, APT)])
            plsc.subcore_barrier()

        one_conv(gh_hbm, rowh, colh, outh)
        one_conv(gt_hbm, rowt, colt, outt)

    return conv


_conv_l1 = _make_conv(True)
_conv_l2 = _make_conv(False)


def _dinv(d):
    return jnp.where(d > 0.0, lax.rsqrt(jnp.where(d > 0.0, d, 1.0)), 0.0)


def _prep1_body(x_ref, w_ref, deg_ref, c1_ref, gh_ref, gt_ref):
    t = jnp.dot(x_ref[...], w_ref[...], preferred_element_type=jnp.float32,
                precision=lax.Precision.HIGHEST)
    deg = deg_ref[...]
    dh = _dinv(deg[:, 0])[:, None]
    dt = _dinv(deg[:, 1])[:, None]
    c1_ref[...] = t[:, :D_HID]
    gh_ref[...] = t[:, D_HID:2 * D_HID] * dh
    gt_ref[...] = t[:, 2 * D_HID:] * dt


def _combine1_body(c1_ref, ah_ref, at_ref, deg_ref, w_ref,
                   c2_ref, gh2_ref, gt2_ref):
    deg = deg_ref[...]
    dh = _dinv(deg[:, 0])[:, None]
    dt = _dinv(deg[:, 1])[:, None]
    ah = jnp.concatenate([ah_ref[0], ah_ref[1]], axis=1)
    at = jnp.concatenate([at_ref[0], at_ref[1]], axis=1)
    h = c1_ref[...] + BETA * dh * ah + (1.0 - BETA) * dt * at
    h = jnp.maximum(h, 0.0)
    t2 = jnp.dot(h, w_ref[...], preferred_element_type=jnp.float32,
                 precision=lax.Precision.HIGHEST)
    c2_ref[...] = t2[:, :D_OUT]
    gh2_ref[...] = t2[:, D_OUT:2 * D_OUT] * dh
    gt2_ref[...] = t2[:, 2 * D_OUT:] * dt


def _combine2_body(c2_ref, ph_ref, pt_ref, deg_ref, probs_ref, logits_ref):
    deg = deg_ref[...]
    dh = _dinv(deg[:, 0])[:, None]
    dt = _dinv(deg[:, 1])[:, None]
    f = (c2_ref[...]
         + BETA * dh * (ph_ref[0] + ph_ref[1])
         + (1.0 - BETA) * dt * (pt_ref[0] + pt_ref[1]))
    m = jnp.max(f, axis=1, keepdims=True)
    e = jnp.exp(f - m)
    s = jnp.sum(e, axis=1, keepdims=True)
    probs_ref[...] = e / s
    logits_ref[...] = (f - m) - jnp.log(s)


def _row_spec(w):
    return pl.BlockSpec((BLK, w), lambda i: (i, 0))


def _pair_spec(w):
    return pl.BlockSpec((2, BLK, w), lambda i: (0, i, 0))


_deg_spec = pl.BlockSpec((BLK, 2), lambda i: (i, 0))


def _full_spec(h, w):
    return pl.BlockSpec((h, w), lambda i: (0, 0))


def _pad_idx(ei):
    npad = EP - E
    padr = (jnp.arange(npad, dtype=jnp.int32) * 97) % N
    padc = N + jnp.arange(npad, dtype=jnp.int32) % (NA - N)
    rows = jnp.concatenate([ei[0], padr]).reshape(NCHUNK, CH)
    cols = jnp.concatenate([ei[1], padc]).reshape(NCHUNK, CH)
    return rows, cols


def kernel(x, homo_edge_index, hetero_edge_index,
           W_center1, W_homo1, W_hetero1, W_center2, W_homo2, W_hetero2):
    rh, ch = _pad_idx(homo_edge_index)
    rt, ct = _pad_idx(hetero_edge_index)

    deg = _deg_kernel(ch, ct).T

    w1 = jnp.concatenate([W_center1, W_homo1, W_hetero1], axis=1)
    c1, gh1, gt1 = pl.pallas_call(
        _prep1_body,
        grid=(GRID,),
        in_specs=[_row_spec(D_IN), _full_spec(D_IN, 3 * D_HID), _deg_spec],
        out_specs=[_row_spec(D_HID)] * 3,
        out_shape=[jax.ShapeDtypeStruct((N, D_HID), jnp.float32)] * 3,
    )(x, w1, deg)

    a_h1, a_t1 = _conv_l1(gh1.reshape(2 * N, 128), gt1.reshape(2 * N, 128),
                          rh, ch, rt, ct)

    w2 = jnp.concatenate([W_center2, W_homo2, W_hetero2], axis=1)
    c2, gh2, gt2 = pl.pallas_call(
        _combine1_body,
        grid=(GRID,),
        in_specs=[_row_spec(D_HID), _pair_spec(128), _pair_spec(128),
                  _deg_spec, _full_spec(D_HID, 3 * D_OUT)],
        out_specs=[_row_spec(D_OUT)] * 3,
        out_shape=[jax.ShapeDtypeStruct((N, D_OUT), jnp.float32)] * 3,
    )(c1, a_h1, a_t1, deg, w2)

    p_h2, p_t2 = _conv_l2(gh2, gt2, rh, ch, rt, ct)

    probs, logits = pl.pallas_call(
        _combine2_body,
        grid=(GRID,),
        in_specs=[_row_spec(D_OUT), _pair_spec(D_OUT), _pair_spec(D_OUT),
                  _deg_spec],
        out_specs=[_row_spec(D_OUT)] * 2,
        out_shape=[jax.ShapeDtypeStruct((N, D_OUT), jnp.float32)] * 2,
    )(c2, p_h2, p_t2, deg)

    return (probs, logits)


# speedup vs baseline: 13.1306x; 1.0001x over previous
import functools

import jax
import jax.numpy as jnp
from jax import lax
from jax.experimental import pallas as pl
from jax.experimental.pallas import tpu as pltpu
from jax.experimental.pallas import tpu_sc as plsc

N = 10000
E = 160000
D_IN = 256
D_HID = 256
D_OUT = 128
BETA = 0.5

NC = 2
NS = 16
L = 16

CH = 128
NCHUNK = 1280
EP = NCHUNK * CH
NA = 10240
APT = NA // NS
WB = N // NS
NW = 40
BLK = 2000
GRID = N // BLK

_mesh = plsc.VectorSubcoreMesh(core_axis_name="c", subcore_axis_name="s")


def _fill(ref, n, vec):
    def body(i, _):
        ref[pl.ds(i * L, L)] = vec
        return 0
    lax.fori_loop(0, n // L, body, 0)


@functools.partial(
    pl.kernel,
    out_type=jax.ShapeDtypeStruct((NC, NA), jnp.float32),
    mesh=_mesh,
    scratch_types=[
        pltpu.VMEM((NCHUNK // NS, CH), jnp.int32),
        pltpu.VMEM((CH,), jnp.float32),
        pltpu.VMEM((APT,), jnp.float32),
        pltpu.VMEM_SHARED((NA,), jnp.float32),
    ],
)
def _deg_kernel(colh_hbm, colt_hbm, deg_out, idx_v, ones_v, zeros_v, acc):
    c = lax.axis_index("c")
    t = lax.axis_index("s")
    cpt = NCHUNK // NS
    _fill(ones_v, CH, jnp.ones((L,), jnp.float32))
    _fill(zeros_v, APT, jnp.zeros((L,), jnp.float32))
    pltpu.sync_copy(zeros_v, acc.at[pl.ds(t * APT, APT)])

    @pl.when(c == 0)
    def _():
        pltpu.sync_copy(colh_hbm.at[pl.ds(t * cpt, cpt)], idx_v)

    @pl.when(c == 1)
    def _():
        pltpu.sync_copy(colt_hbm.at[pl.ds(t * cpt, cpt)], idx_v)

    plsc.subcore_barrier()

    def body(j, _):
        pltpu.sync_copy(ones_v, acc.at[idx_v.at[j]], add=True)
        return 0
    lax.fori_loop(0, cpt, body, 0)

    plsc.subcore_barrier()
    pltpu.sync_copy(acc.at[pl.ds(t * APT, APT)],
                    deg_out.at[c, pl.ds(t * APT, APT)])


def _make_conv(split_features):
    cpt = NCHUNK // NS if split_features else NCHUNK // (NC * NS)

    @functools.partial(
        pl.kernel,
        out_type=[jax.ShapeDtypeStruct((NC, NA, 128), jnp.float32)] * 2,
        mesh=_mesh,
        scratch_types=[
            pltpu.VMEM((NW, CH), jnp.int32),
            pltpu.VMEM((NW, CH), jnp.int32),
            pltpu.VMEM((CH, 128), jnp.float32),
            pltpu.VMEM((CH, 128), jnp.float32),
            pltpu.VMEM_SHARED((NA, 128), jnp.float32),
            pltpu.SemaphoreType.DMA,
            pltpu.SemaphoreType.DMA,
        ],
    )
    def conv(gh_hbm, gt_hbm, rowh, colh, rowt, colt, outh, outt,
             idxr, idxc, bufa, bufb, acc, sema, semb):
        c = lax.axis_index("c")
        t = lax.axis_index("s")

        base = t * cpt if split_features else c * (NCHUNK // NC) + t * cpt

        def one_conv(g_hbm, row_hbm, col_hbm, out_hbm):
            def zb(i, _):
                bufa[i // 8, pl.ds((i % 8) * L, L)] = jnp.zeros((L,),
                                                                jnp.float32)
                return 0
            lax.fori_loop(0, CH * 8, zb, 0)
            for z in range(APT // CH):
                pltpu.sync_copy(bufa, acc.at[pl.ds(t * APT + z * CH, CH)])
            plsc.subcore_barrier()

            for w in range(cpt // NW):
                wbase = base + w * NW
                pltpu.sync_copy(row_hbm.at[pl.ds(wbase, NW)], idxr)
                pltpu.sync_copy(col_hbm.at[pl.ds(wbase, NW)], idxc)
                if split_features:
                    def tr(i, _):
                        j = i // 8
                        k = i % 8
                        v = idxr[j, pl.ds(k * L, L)]
                        idxr[j, pl.ds(k * L, L)] = v * 2 + c
                        return 0
                    lax.fori_loop(0, NW * 8, tr, 0)

                pltpu.async_copy(g_hbm.at[idxr.at[0]], bufa, sema)

                def step(jj, _):
                    c0 = 2 * jj
                    c1 = 2 * jj + 1
                    pltpu.make_async_copy(g_hbm.at[idxr.at[0]], bufa,
                                          sema).wait()
                    pltpu.async_copy(g_hbm.at[idxr.at[c1]], bufb, semb)
                    pltpu.sync_copy(bufa, acc.at[idxc.at[c0]], add=True)
                    pltpu.make_async_copy(g_hbm.at[idxr.at[0]], bufb,
                                          semb).wait()
                    nxt = jnp.minimum(c0 + 2, NW - 1)
                    pltpu.async_copy(g_hbm.at[idxr.at[nxt]], bufa, sema)
                    pltpu.sync_copy(bufb, acc.at[idxc.at[c1]], add=True)
                    return 0
                lax.fori_loop(0, NW // 2, step, 0)
                pltpu.make_async_copy(g_hbm.at[idxr.at[0]], bufa, sema).wait()

            plsc.subcore_barrier()
            pltpu.sync_copy(acc.at[pl.ds(t * APT, APT)],
                            out_hbm.at[c, pl.ds(t * APT, APT)])
            plsc.subcore_barrier()

        one_conv(gh_hbm, rowh, colh, outh)
        one_conv(gt_hbm, rowt, colt, outt)

    return conv


_conv_l1 = _make_conv(True)
_conv_l2 = _make_conv(False)


def _dinv(d):
    return jnp.where(d > 0.0, lax.rsqrt(jnp.where(d > 0.0, d, 1.0)), 0.0)


def _prep1_body(x_ref, w_ref, deg_ref, c1_ref, gh_ref, gt_ref):
    t = jnp.dot(x_ref[...], w_ref[...], preferred_element_type=jnp.float32,
                precision=lax.Precision.HIGHEST)
    deg = deg_ref[...]
    dh = _dinv(deg[:, 0])[:, None]
    dt = _dinv(deg[:, 1])[:, None]
    c1_ref[...] = t[:, :D_HID]
    gh_ref[...] = t[:, D_HID:2 * D_HID] * dh
    gt_ref[...] = t[:, 2 * D_HID:] * dt


def _combine1_body(c1_ref, ah_ref, at_ref, deg_ref, w_ref,
                   c2_ref, gh2_ref, gt2_ref):
    deg = deg_ref[...]
    dh = _dinv(deg[:, 0])[:, None]
    dt = _dinv(deg[:, 1])[:, None]
    ah = jnp.concatenate([ah_ref[0], ah_ref[1]], axis=1)
    at = jnp.concatenate([at_ref[0], at_ref[1]], axis=1)
    h = c1_ref[...] + BETA * dh * ah + (1.0 - BETA) * dt * at
    h = jnp.maximum(h, 0.0)
    t2 = jnp.dot(h, w_ref[...], preferred_element_type=jnp.float32,
                 precision=lax.Precision.HIGHEST)
    c2_ref[...] = t2[:, :D_OUT]
    gh2_ref[...] = t2[:, D_OUT:2 * D_OUT] * dh
    gt2_ref[...] = t2[:, 2 * D_OUT:] * dt


def _combine2_body(c2_ref, ph_ref, pt_ref, deg_ref, probs_ref, logits_ref):
    deg = deg_ref[...]
    dh = _dinv(deg[:, 0])[:, None]
    dt = _dinv(deg[:, 1])[:, None]
    f = (c2_ref[...]
         + BETA * dh * (ph_ref[0] + ph_ref[1])
         + (1.0 - BETA) * dt * (pt_ref[0] + pt_ref[1]))
    m = jnp.max(f, axis=1, keepdims=True)
    e = jnp.exp(f - m)
    s = jnp.sum(e, axis=1, keepdims=True)
    probs_ref[...] = e / s
    logits_ref[...] = (f - m) - jnp.log(s)


def _row_spec(w):
    return pl.BlockSpec((BLK, w), lambda i: (i, 0))


def _pair_spec(w):
    return pl.BlockSpec((2, BLK, w), lambda i: (0, i, 0))


_deg_spec = pl.BlockSpec((BLK, 2), lambda i: (i, 0))


def _full_spec(h, w):
    return pl.BlockSpec((h, w), lambda i: (0, 0))


def _pad_idx(ei):
    npad = EP - E
    padr = (jnp.arange(npad, dtype=jnp.int32) * 97) % N
    padc = N + jnp.arange(npad, dtype=jnp.int32) % (NA - N)
    rows = jnp.concatenate([ei[0], padr]).reshape(NCHUNK, CH)
    cols = jnp.concatenate([ei[1], padc]).reshape(NCHUNK, CH)
    return rows, cols


def kernel(x, homo_edge_index, hetero_edge_index,
           W_center1, W_homo1, W_hetero1, W_center2, W_homo2, W_hetero2):
    rh, ch = _pad_idx(homo_edge_index)
    rt, ct = _pad_idx(hetero_edge_index)

    deg = _deg_kernel(ch, ct).T

    w1 = jnp.concatenate([W_center1, W_homo1, W_hetero1], axis=1)
    c1, gh1, gt1 = pl.pallas_call(
        _prep1_body,
        grid=(GRID,),
        in_specs=[_row_spec(D_IN), _full_spec(D_IN, 3 * D_HID), _deg_spec],
        out_specs=[_row_spec(D_HID)] * 3,
        out_shape=[jax.ShapeDtypeStruct((N, D_HID), jnp.float32)] * 3,
    )(x, w1, deg)

    a_h1, a_t1 = _conv_l1(gh1.reshape(2 * N, 128), gt1.reshape(2 * N, 128),
                          rh, ch, rt, ct)

    w2 = jnp.concatenate([W_center2, W_homo2, W_hetero2], axis=1)
    c2, gh2, gt2 = pl.pallas_call(
        _combine1_body,
        grid=(GRID,),
        in_specs=[_row_spec(D_HID), _pair_spec(128), _pair_spec(128),
                  _deg_spec, _full_spec(D_HID, 3 * D_OUT)],
        out_specs=[_row_spec(D_OUT)] * 3,
        out_shape=[jax.ShapeDtypeStruct((N, D_OUT), jnp.float32)] * 3,
    )(c1, a_h1, a_t1, deg, w2)

    p_h2, p_t2 = _conv_l2(gh2, gt2, rh, ch, rt, ct)

    probs, logits = pl.pallas_call(
        _combine2_body,
        grid=(GRID,),
        in_specs=[_row_spec(D_OUT), _pair_spec(D_OUT), _pair_spec(D_OUT),
                  _deg_spec],
        out_specs=[_row_spec(D_OUT)] * 2,
        out_shape=[jax.ShapeDtypeStruct((N, D_OUT), jnp.float32)] * 2,
    )(c2, p_h2, p_t2, deg)

    return (probs, logits)


# speedup vs baseline: 13.3365x; 1.0157x over previous
import functools

import jax
import jax.numpy as jnp
from jax import lax
from jax.experimental import pallas as pl
from jax.experimental.pallas import tpu as pltpu
from jax.experimental.pallas import tpu_sc as plsc

N = 10000
E = 160000
D_IN = 256
D_HID = 256
D_OUT = 128
BETA = 0.5

NC = 2
NS = 16
L = 16

CH = 125
NCHUNK = 1280
NA = 10240
APT = NA // NS
WB = N // NS
NW = 40
BLK = 2000
GRID = N // BLK

_mesh = plsc.VectorSubcoreMesh(core_axis_name="c", subcore_axis_name="s")


def _fill(ref, n, vec):
    def body(i, _):
        ref[pl.ds(i * L, L)] = vec
        return 0
    lax.fori_loop(0, n // L, body, 0)


@functools.partial(
    pl.kernel,
    out_type=jax.ShapeDtypeStruct((NC, NA), jnp.float32),
    mesh=_mesh,
    scratch_types=[
        pltpu.VMEM((NCHUNK // NS, CH), jnp.int32),
        pltpu.VMEM((128,), jnp.float32),
        pltpu.VMEM((APT,), jnp.float32),
        pltpu.VMEM_SHARED((NA,), jnp.float32),
    ],
)
def _deg_kernel(colh_hbm, colt_hbm, deg_out, idx_v, ones_v, zeros_v, acc):
    c = lax.axis_index("c")
    t = lax.axis_index("s")
    cpt = NCHUNK // NS
    _fill(ones_v, 128, jnp.ones((L,), jnp.float32))
    _fill(zeros_v, APT, jnp.zeros((L,), jnp.float32))
    pltpu.sync_copy(zeros_v, acc.at[pl.ds(t * APT, APT)])

    @pl.when(c == 0)
    def _():
        pltpu.sync_copy(colh_hbm.at[pl.ds(t * cpt, cpt)], idx_v)

    @pl.when(c == 1)
    def _():
        pltpu.sync_copy(colt_hbm.at[pl.ds(t * cpt, cpt)], idx_v)

    plsc.subcore_barrier()

    def body(j, _):
        pltpu.sync_copy(ones_v.at[pl.ds(0, CH)], acc.at[idx_v.at[j]],
                        add=True)
        return 0
    lax.fori_loop(0, cpt, body, 0)

    plsc.subcore_barrier()
    pltpu.sync_copy(acc.at[pl.ds(t * APT, APT)],
                    deg_out.at[c, pl.ds(t * APT, APT)])


def _make_conv(split_features):
    cpt = NCHUNK // NS if split_features else NCHUNK // (NC * NS)

    @functools.partial(
        pl.kernel,
        out_type=[jax.ShapeDtypeStruct((NC, NA, 128), jnp.float32)] * 2,
        mesh=_mesh,
        scratch_types=[
            pltpu.VMEM((NW, CH), jnp.int32),
            pltpu.VMEM((NW, CH), jnp.int32),
            pltpu.VMEM((CH, 128), jnp.float32),
            pltpu.VMEM((CH, 128), jnp.float32),
            pltpu.VMEM_SHARED((NA, 128), jnp.float32),
            pltpu.SemaphoreType.DMA,
            pltpu.SemaphoreType.DMA,
        ],
    )
    def conv(gh_hbm, gt_hbm, rowh0, rowh1, colh, rowt0, rowt1, colt,
             outh, outt, idxr, idxc, bufa, bufb, acc, sema, semb):
        c = lax.axis_index("c")
        t = lax.axis_index("s")

        base = t * cpt if split_features else c * (NCHUNK // NC) + t * cpt

        def one_conv(g_hbm, row0_hbm, row1_hbm, col_hbm, out_hbm):
            def zb(i, _):
                bufa[i // 8, pl.ds((i % 8) * L, L)] = jnp.zeros((L,),
                                                                jnp.float32)
                return 0
            lax.fori_loop(0, CH * 8, zb, 0)
            for z in range(APT // CH):
                pltpu.sync_copy(bufa, acc.at[pl.ds(t * APT + z * CH, CH)])
            plsc.subcore_barrier()

            for w in range(cpt // NW):
                wbase = base + w * NW

                @pl.when(c == 0)
                def _():
                    pltpu.sync_copy(row0_hbm.at[pl.ds(wbase, NW)], idxr)

                @pl.when(c == 1)
                def _():
                    pltpu.sync_copy(row1_hbm.at[pl.ds(wbase, NW)], idxr)

                pltpu.sync_copy(col_hbm.at[pl.ds(wbase, NW)], idxc)

                pltpu.async_copy(g_hbm.at[idxr.at[0]], bufa, sema)

                def step(jj, _):
                    c0 = 2 * jj
                    c1 = 2 * jj + 1
                    pltpu.make_async_copy(g_hbm.at[idxr.at[0]], bufa,
                                          sema).wait()
                    pltpu.async_copy(g_hbm.at[idxr.at[c1]], bufb, semb)
                    pltpu.sync_copy(bufa, acc.at[idxc.at[c0]], add=True)
                    pltpu.make_async_copy(g_hbm.at[idxr.at[0]], bufb,
                                          semb).wait()
                    nxt = jnp.minimum(c0 + 2, NW - 1)
                    pltpu.async_copy(g_hbm.at[idxr.at[nxt]], bufa, sema)
                    pltpu.sync_copy(bufb, acc.at[idxc.at[c1]], add=True)
                    return 0
                lax.fori_loop(0, NW // 2, step, 0)
                pltpu.make_async_copy(g_hbm.at[idxr.at[0]], bufa, sema).wait()

            plsc.subcore_barrier()
            pltpu.sync_copy(acc.at[pl.ds(t * APT, APT)],
                            out_hbm.at[c, pl.ds(t * APT, APT)])
            plsc.subcore_barrier()

        one_conv(gh_hbm, rowh0, rowh1, colh, outh)
        one_conv(gt_hbm, rowt0, rowt1, colt, outt)

    return conv


_conv_l1 = _make_conv(True)
_conv_l2 = _make_conv(False)


def _dinv(d):
    return jnp.where(d > 0.0, lax.rsqrt(jnp.where(d > 0.0, d, 1.0)), 0.0)


def _prep1_body(x_ref, w_ref, deg_ref, c1_ref, gh_ref, gt_ref):
    t = jnp.dot(x_ref[...], w_ref[...], preferred_element_type=jnp.float32,
                precision=lax.Precision.HIGHEST)
    deg = deg_ref[...]
    dh = _dinv(deg[:, 0])[:, None]
    dt = _dinv(deg[:, 1])[:, None]
    c1_ref[...] = t[:, :D_HID]
    gh_ref[...] = t[:, D_HID:2 * D_HID] * dh
    gt_ref[...] = t[:, 2 * D_HID:] * dt


def _combine1_body(c1_ref, ah_ref, at_ref, deg_ref, w_ref,
                   c2_ref, gh2_ref, gt2_ref):
    deg = deg_ref[...]
    dh = _dinv(deg[:, 0])[:, None]
    dt = _dinv(deg[:, 1])[:, None]
    ah = jnp.concatenate([ah_ref[0], ah_ref[1]], axis=1)
    at = jnp.concatenate([at_ref[0], at_ref[1]], axis=1)
    h = c1_ref[...] + BETA * dh * ah + (1.0 - BETA) * dt * at
    h = jnp.maximum(h, 0.0)
    t2 = jnp.dot(h, w_ref[...], preferred_element_type=jnp.float32,
                 precision=lax.Precision.HIGHEST)
    c2_ref[...] = t2[:, :D_OUT]
    gh2_ref[...] = t2[:, D_OUT:2 * D_OUT] * dh
    gt2_ref[...] = t2[:, 2 * D_OUT:] * dt


def _combine2_body(c2_ref, ph_ref, pt_ref, deg_ref, probs_ref, logits_ref):
    deg = deg_ref[...]
    dh = _dinv(deg[:, 0])[:, None]
    dt = _dinv(deg[:, 1])[:, None]
    f = (c2_ref[...]
         + BETA * dh * (ph_ref[0] + ph_ref[1])
         + (1.0 - BETA) * dt * (pt_ref[0] + pt_ref[1]))
    m = jnp.max(f, axis=1, keepdims=True)
    e = jnp.exp(f - m)
    s = jnp.sum(e, axis=1, keepdims=True)
    probs_ref[...] = e / s
    logits_ref[...] = (f - m) - jnp.log(s)


def _row_spec(w):
    return pl.BlockSpec((BLK, w), lambda i: (i, 0))


def _pair_spec(w):
    return pl.BlockSpec((2, BLK, w), lambda i: (0, i, 0))


_deg_spec = pl.BlockSpec((BLK, 2), lambda i: (i, 0))


def _full_spec(h, w):
    return pl.BlockSpec((h, w), lambda i: (0, 0))


def _chunk_idx(ei):
    rows = ei[0].reshape(NCHUNK, CH)
    cols = ei[1].reshape(NCHUNK, CH)
    return rows, cols


def kernel(x, homo_edge_index, hetero_edge_index,
           W_center1, W_homo1, W_hetero1, W_center2, W_homo2, W_hetero2):
    rh, ch = _chunk_idx(homo_edge_index)
    rt, ct = _chunk_idx(hetero_edge_index)
    rh2, rt2 = rh * 2, rt * 2

    deg = _deg_kernel(ch, ct).T

    w1 = jnp.concatenate([W_center1, W_homo1, W_hetero1], axis=1)
    c1, gh1, gt1 = pl.pallas_call(
        _prep1_body,
        grid=(GRID,),
        in_specs=[_row_spec(D_IN), _full_spec(D_IN, 3 * D_HID), _deg_spec],
        out_specs=[_row_spec(D_HID)] * 3,
        out_shape=[jax.ShapeDtypeStruct((N, D_HID), jnp.float32)] * 3,
    )(x, w1, deg)

    a_h1, a_t1 = _conv_l1(gh1.reshape(2 * N, 128), gt1.reshape(2 * N, 128),
                          rh2, rh2 + 1, ch, rt2, rt2 + 1, ct)

    w2 = jnp.concatenate([W_center2, W_homo2, W_hetero2], axis=1)
    c2, gh2, gt2 = pl.pallas_call(
        _combine1_body,
        grid=(GRID,),
        in_specs=[_row_spec(D_HID), _pair_spec(128), _pair_spec(128),
                  _deg_spec, _full_spec(D_HID, 3 * D_OUT)],
        out_specs=[_row_spec(D_OUT)] * 3,
        out_shape=[jax.ShapeDtypeStruct((N, D_OUT), jnp.float32)] * 3,
    )(c1, a_h1, a_t1, deg, w2)

    p_h2, p_t2 = _conv_l2(gh2, gt2, rh, rh, ch, rt, rt, ct)

    probs, logits = pl.pallas_call(
        _combine2_body,
        grid=(GRID,),
        in_specs=[_row_spec(D_OUT), _pair_spec(D_OUT), _pair_spec(D_OUT),
                  _deg_spec],
        out_specs=[_row_spec(D_OUT)] * 2,
        out_shape=[jax.ShapeDtypeStruct((N, D_OUT), jnp.float32)] * 2,
    )(c2, p_h2, p_t2, deg)

    return (probs, logits)


# speedup vs baseline: 13.4975x; 1.0121x over previous
import functools

import jax
import jax.numpy as jnp
from jax import lax
from jax.experimental import pallas as pl
from jax.experimental.pallas import tpu as pltpu
from jax.experimental.pallas import tpu_sc as plsc

N = 10000
E = 160000
D_IN = 256
D_HID = 256
D_OUT = 128
BETA = 0.5

NC = 2
NS = 16
L = 16

CH = 128
NCHUNK = 1280
EP = NCHUNK * CH
NA = 10240
APT = NA // NS
WB = N // NS
NW = 40
BLK = 2000
GRID = N // BLK

_mesh = plsc.VectorSubcoreMesh(core_axis_name="c", subcore_axis_name="s")


def _fill(ref, n, vec):
    def body(i, _):
        ref[pl.ds(i * L, L)] = vec
        return 0
    lax.fori_loop(0, n // L, body, 0)


@functools.partial(
    pl.kernel,
    out_type=jax.ShapeDtypeStruct((NC, NA), jnp.float32),
    mesh=_mesh,
    scratch_types=[
        pltpu.VMEM((NCHUNK // NS, CH), jnp.int32),
        pltpu.VMEM((CH,), jnp.float32),
        pltpu.VMEM((APT,), jnp.float32),
        pltpu.VMEM_SHARED((NA,), jnp.float32),
    ],
)
def _deg_kernel(colh_hbm, colt_hbm, deg_out, idx_v, ones_v, zeros_v, acc):
    c = lax.axis_index("c")
    t = lax.axis_index("s")
    cpt = NCHUNK // NS
    _fill(ones_v, CH, jnp.ones((L,), jnp.float32))
    _fill(zeros_v, APT, jnp.zeros((L,), jnp.float32))
    pltpu.sync_copy(zeros_v, acc.at[pl.ds(t * APT, APT)])

    @pl.when(c == 0)
    def _():
        pltpu.sync_copy(colh_hbm.at[pl.ds(t * cpt, cpt)], idx_v)

    @pl.when(c == 1)
    def _():
        pltpu.sync_copy(colt_hbm.at[pl.ds(t * cpt, cpt)], idx_v)

    plsc.subcore_barrier()

    def body(j, _):
        pltpu.sync_copy(ones_v, acc.at[idx_v.at[j]], add=True)
        return 0
    lax.fori_loop(0, cpt, body, 0)

    plsc.subcore_barrier()
    pltpu.sync_copy(acc.at[pl.ds(t * APT, APT)],
                    deg_out.at[c, pl.ds(t * APT, APT)])


def _make_conv(split_features):
    cpt = NCHUNK // NS if split_features else NCHUNK // (NC * NS)

    @functools.partial(
        pl.kernel,
        out_type=[jax.ShapeDtypeStruct((NC, NA, 128), jnp.float32)] * 2,
        mesh=_mesh,
        scratch_types=[
            pltpu.VMEM((NW, CH), jnp.int32),
            pltpu.VMEM((NW, CH), jnp.int32),
            pltpu.VMEM((CH, 128), jnp.float32),
            pltpu.VMEM((CH, 128), jnp.float32),
            pltpu.VMEM_SHARED((NA, 128), jnp.float32),
            pltpu.SemaphoreType.DMA,
            pltpu.SemaphoreType.DMA,
        ],
    )
    def conv(gh_hbm, gt_hbm, rowh0, rowh1, colh, rowt0, rowt1, colt,
             outh, outt, idxr, idxc, bufa, bufb, acc, sema, semb):
        c = lax.axis_index("c")
        t = lax.axis_index("s")

        base = t * cpt if split_features else c * (NCHUNK // NC) + t * cpt

        def one_conv(g_hbm, row0_hbm, row1_hbm, col_hbm, out_hbm):
            def zb(i, _):
                bufa[i // 8, pl.ds((i % 8) * L, L)] = jnp.zeros((L,),
                                                                jnp.float32)
                return 0
            lax.fori_loop(0, CH * 8, zb, 0)
            for z in range(APT // CH):
                pltpu.sync_copy(bufa, acc.at[pl.ds(t * APT + z * CH, CH)])
            plsc.subcore_barrier()

            for w in range(cpt // NW):
                wbase = base + w * NW

                @pl.when(c == 0)
                def _():
                    pltpu.sync_copy(row0_hbm.at[pl.ds(wbase, NW)], idxr)

                @pl.when(c == 1)
                def _():
                    pltpu.sync_copy(row1_hbm.at[pl.ds(wbase, NW)], idxr)

                pltpu.sync_copy(col_hbm.at[pl.ds(wbase, NW)], idxc)

                pltpu.async_copy(g_hbm.at[idxr.at[0]], bufa, sema)

                def step(jj, _):
                    c0 = 2 * jj
                    c1 = 2 * jj + 1
                    pltpu.make_async_copy(g_hbm.at[idxr.at[0]], bufa,
                                          sema).wait()
                    pltpu.async_copy(g_hbm.at[idxr.at[c1]], bufb, semb)
                    pltpu.sync_copy(bufa, acc.at[idxc.at[c0]], add=True)
                    pltpu.make_async_copy(g_hbm.at[idxr.at[0]], bufb,
                                          semb).wait()
                    nxt = jnp.minimum(c0 + 2, NW - 1)
                    pltpu.async_copy(g_hbm.at[idxr.at[nxt]], bufa, sema)
                    pltpu.sync_copy(bufb, acc.at[idxc.at[c1]], add=True)
                    return 0
                lax.fori_loop(0, NW // 2, step, 0)
                pltpu.make_async_copy(g_hbm.at[idxr.at[0]], bufa, sema).wait()

            plsc.subcore_barrier()
            pltpu.sync_copy(acc.at[pl.ds(t * APT, APT)],
                            out_hbm.at[c, pl.ds(t * APT, APT)])
            plsc.subcore_barrier()

        one_conv(gh_hbm, rowh0, rowh1, colh, outh)
        one_conv(gt_hbm, rowt0, rowt1, colt, outt)

    return conv


_conv_l1 = _make_conv(True)
_conv_l2 = _make_conv(False)


def _dinv(d):
    return jnp.where(d > 0.0, lax.rsqrt(jnp.where(d > 0.0, d, 1.0)), 0.0)


def _prep1_body(x_ref, w_ref, deg_ref, c1_ref, gh_ref, gt_ref):
    t = jnp.dot(x_ref[...], w_ref[...], preferred_element_type=jnp.float32,
                precision=lax.Precision.HIGHEST)
    deg = deg_ref[...]
    dh = _dinv(deg[:, 0])[:, None]
    dt = _dinv(deg[:, 1])[:, None]
    c1_ref[...] = t[:, :D_HID]
    gh_ref[...] = t[:, D_HID:2 * D_HID] * dh
    gt_ref[...] = t[:, 2 * D_HID:] * dt


def _combine1_body(c1_ref, ah_ref, at_ref, deg_ref, w_ref,
                   c2_ref, gh2_ref, gt2_ref):
    deg = deg_ref[...]
    dh = _dinv(deg[:, 0])[:, None]
    dt = _dinv(deg[:, 1])[:, None]
    ah = jnp.concatenate([ah_ref[0], ah_ref[1]], axis=1)
    at = jnp.concatenate([at_ref[0], at_ref[1]], axis=1)
    h = c1_ref[...] + BETA * dh * ah + (1.0 - BETA) * dt * at
    h = jnp.maximum(h, 0.0)
    t2 = jnp.dot(h, w_ref[...], preferred_element_type=jnp.float32,
                 precision=lax.Precision.HIGHEST)
    c2_ref[...] = t2[:, :D_OUT]
    gh2_ref[...] = t2[:, D_OUT:2 * D_OUT] * dh
    gt2_ref[...] = t2[:, 2 * D_OUT:] * dt


def _combine2_body(c2_ref, ph_ref, pt_ref, deg_ref, probs_ref, logits_ref):
    deg = deg_ref[...]
    dh = _dinv(deg[:, 0])[:, None]
    dt = _dinv(deg[:, 1])[:, None]
    f = (c2_ref[...]
         + BETA * dh * (ph_ref[0] + ph_ref[1])
         + (1.0 - BETA) * dt * (pt_ref[0] + pt_ref[1]))
    m = jnp.max(f, axis=1, keepdims=True)
    e = jnp.exp(f - m)
    s = jnp.sum(e, axis=1, keepdims=True)
    probs_ref[...] = e / s
    logits_ref[...] = (f - m) - jnp.log(s)


def _row_spec(w):
    return pl.BlockSpec((BLK, w), lambda i: (i, 0))


def _pair_spec(w):
    return pl.BlockSpec((2, BLK, w), lambda i: (0, i, 0))


_deg_spec = pl.BlockSpec((BLK, 2), lambda i: (i, 0))


def _full_spec(h, w):
    return pl.BlockSpec((h, w), lambda i: (0, 0))


def _pad_idx(ei):
    npad = EP - E
    padr = (jnp.arange(npad, dtype=jnp.int32) * 97) % N
    padc = N + jnp.arange(npad, dtype=jnp.int32) % (NA - N)
    rows = jnp.concatenate([ei[0], padr]).reshape(NCHUNK, CH)
    cols = jnp.concatenate([ei[1], padc]).reshape(NCHUNK, CH)
    return rows, cols


def kernel(x, homo_edge_index, hetero_edge_index,
           W_center1, W_homo1, W_hetero1, W_center2, W_homo2, W_hetero2):
    rh, ch = _pad_idx(homo_edge_index)
    rt, ct = _pad_idx(hetero_edge_index)
    rh2, rt2 = rh * 2, rt * 2

    deg = _deg_kernel(ch, ct).T

    w1 = jnp.concatenate([W_center1, W_homo1, W_hetero1], axis=1)
    c1, gh1, gt1 = pl.pallas_call(
        _prep1_body,
        grid=(GRID,),
        in_specs=[_row_spec(D_IN), _full_spec(D_IN, 3 * D_HID), _deg_spec],
        out_specs=[_row_spec(D_HID)] * 3,
        out_shape=[jax.ShapeDtypeStruct((N, D_HID), jnp.float32)] * 3,
    )(x, w1, deg)

    a_h1, a_t1 = _conv_l1(gh1.reshape(2 * N, 128), gt1.reshape(2 * N, 128),
                          rh2, rh2 + 1, ch, rt2, rt2 + 1, ct)

    w2 = jnp.concatenate([W_center2, W_homo2, W_hetero2], axis=1)
    c2, gh2, gt2 = pl.pallas_call(
        _combine1_body,
        grid=(GRID,),
        in_specs=[_row_spec(D_HID), _pair_spec(128), _pair_spec(128),
                  _deg_spec, _full_spec(D_HID, 3 * D_OUT)],
        out_specs=[_row_spec(D_OUT)] * 3,
        out_shape=[jax.ShapeDtypeStruct((N, D_OUT), jnp.float32)] * 3,
    )(c1, a_h1, a_t1, deg, w2)

    p_h2, p_t2 = _conv_l2(gh2, gt2, rh, rh, ch, rt, rt, ct)

    probs, logits = pl.pallas_call(
        _combine2_body,
        grid=(GRID,),
        in_specs=[_row_spec(D_OUT), _pair_spec(D_OUT), _pair_spec(D_OUT),
                  _deg_spec],
        out_specs=[_row_spec(D_OUT)] * 2,
        out_shape=[jax.ShapeDtypeStruct((N, D_OUT), jnp.float32)] * 2,
    )(c2, p_h2, p_t2, deg)

    return (probs, logits)


# speedup vs baseline: 13.9953x; 1.0369x over previous
import functools

import jax
import jax.numpy as jnp
from jax import lax
from jax.experimental import pallas as pl
from jax.experimental.pallas import tpu as pltpu
from jax.experimental.pallas import tpu_sc as plsc

N = 10000
E = 160000
D_IN = 256
D_HID = 256
D_OUT = 128
BETA = 0.5

NC = 2
NS = 16
L = 16

CH = 128
NCHUNK = 1280
EP = NCHUNK * CH
NA = 10240
APT = NA // NS
WB = N // NS
NW = 40
BLK = 2000
GRID = N // BLK

_mesh = plsc.VectorSubcoreMesh(core_axis_name="c", subcore_axis_name="s")


def _fill(ref, n, vec):
    def body(i, _):
        ref[pl.ds(i * L, L)] = vec
        return 0
    lax.fori_loop(0, n // L, body, 0)


@functools.partial(
    pl.kernel,
    out_type=jax.ShapeDtypeStruct((NC, NA), jnp.float32),
    mesh=_mesh,
    scratch_types=[
        pltpu.VMEM((NCHUNK // NS, CH), jnp.int32),
        pltpu.VMEM((CH,), jnp.float32),
        pltpu.VMEM((APT,), jnp.float32),
        pltpu.VMEM_SHARED((NA,), jnp.float32),
    ],
)
def _deg_kernel(colh_hbm, colt_hbm, deg_out, idx_v, ones_v, zeros_v, acc):
    c = lax.axis_index("c")
    t = lax.axis_index("s")
    cpt = NCHUNK // NS
    _fill(ones_v, CH, jnp.ones((L,), jnp.float32))
    _fill(zeros_v, APT, jnp.zeros((L,), jnp.float32))
    pltpu.sync_copy(zeros_v, acc.at[pl.ds(t * APT, APT)])

    @pl.when(c == 0)
    def _():
        pltpu.sync_copy(colh_hbm.at[pl.ds(t * cpt, cpt)], idx_v)

    @pl.when(c == 1)
    def _():
        pltpu.sync_copy(colt_hbm.at[pl.ds(t * cpt, cpt)], idx_v)

    plsc.subcore_barrier()

    def body(j, _):
        pltpu.sync_copy(ones_v, acc.at[idx_v.at[j]], add=True)
        return 0
    lax.fori_loop(0, cpt, body, 0)

    plsc.subcore_barrier()
    pltpu.sync_copy(acc.at[pl.ds(t * APT, APT)],
                    deg_out.at[c, pl.ds(t * APT, APT)])


def _make_conv(split_features):
    cpt = NCHUNK // NS if split_features else NCHUNK // (NC * NS)

    @functools.partial(
        pl.kernel,
        out_type=[jax.ShapeDtypeStruct((NC, NA, 128), jnp.float32)] * 2,
        mesh=_mesh,
        scratch_types=[
            pltpu.VMEM((NW, CH), jnp.int32),
            pltpu.VMEM((NW, CH), jnp.int32),
            pltpu.VMEM((CH, 128), jnp.float32),
            pltpu.VMEM((CH, 128), jnp.float32),
            pltpu.VMEM_SHARED((NA, 128), jnp.float32),
            pltpu.SemaphoreType.DMA,
            pltpu.SemaphoreType.DMA,
        ],
    )
    def conv(gh_hbm, gt_hbm, rowh0, rowh1, colh, rowt0, rowt1, colt,
             outh, outt, idxr, idxc, bufa, bufb, acc, sema, semb):
        c = lax.axis_index("c")
        t = lax.axis_index("s")

        base = t * cpt if split_features else c * (NCHUNK // NC) + t * cpt

        def one_conv(g_hbm, row0_hbm, row1_hbm, col_hbm, out_hbm):
            def zb(i, _):
                bufa[i // 8, pl.ds((i % 8) * L, L)] = jnp.zeros((L,),
                                                                jnp.float32)
                return 0
            lax.fori_loop(0, CH * 8, zb, 0)
            for z in range(APT // CH):
                pltpu.sync_copy(bufa, acc.at[pl.ds(t * APT + z * CH, CH)])
            plsc.subcore_barrier()

            for w in range(cpt // NW):
                wbase = base + w * NW

                @pl.when(c == 0)
                def _():
                    pltpu.sync_copy(row0_hbm.at[pl.ds(wbase, NW)], idxr)

                @pl.when(c == 1)
                def _():
                    pltpu.sync_copy(row1_hbm.at[pl.ds(wbase, NW)], idxr)

                pltpu.sync_copy(col_hbm.at[pl.ds(wbase, NW)], idxc)

                pltpu.async_copy(g_hbm.at[idxr.at[0]], bufa, sema)

                def step(jj, _):
                    c0 = 2 * jj
                    c1 = 2 * jj + 1
                    pltpu.make_async_copy(g_hbm.at[idxr.at[0]], bufa,
                                          sema).wait()
                    pltpu.async_copy(g_hbm.at[idxr.at[c1]], bufb, semb)
                    pltpu.sync_copy(bufa, acc.at[idxc.at[c0]], add=True)
                    pltpu.make_async_copy(g_hbm.at[idxr.at[0]], bufb,
                                          semb).wait()
                    nxt = jnp.minimum(c0 + 2, NW - 1)
                    pltpu.async_copy(g_hbm.at[idxr.at[nxt]], bufa, sema)
                    pltpu.sync_copy(bufb, acc.at[idxc.at[c1]], add=True)
                    return 0
                lax.fori_loop(0, NW // 2, step, 0)
                pltpu.make_async_copy(g_hbm.at[idxr.at[0]], bufa, sema).wait()

            plsc.subcore_barrier()
            pltpu.sync_copy(acc.at[pl.ds(t * APT, APT)],
                            out_hbm.at[c, pl.ds(t * APT, APT)])
            plsc.subcore_barrier()

        one_conv(gh_hbm, rowh0, rowh1, colh, outh)
        one_conv(gt_hbm, rowt0, rowt1, colt, outt)

    return conv


_conv_l1 = _make_conv(True)
_conv_l2 = _make_conv(False)


def _dinv(d):
    return jnp.where(d > 0.0, lax.rsqrt(jnp.where(d > 0.0, d, 1.0)), 0.0)


def _prep1_body(x_ref, w_ref, deg_ref, c1_ref, gh_ref, gt_ref):
    t = jnp.dot(x_ref[...], w_ref[...], preferred_element_type=jnp.float32,
                precision=lax.Precision.DEFAULT)
    deg = deg_ref[...]
    dh = _dinv(deg[:, 0])[:, None]
    dt = _dinv(deg[:, 1])[:, None]
    c1_ref[...] = t[:, :D_HID]
    gh_ref[...] = t[:, D_HID:2 * D_HID] * dh
    gt_ref[...] = t[:, 2 * D_HID:] * dt


def _combine1_body(c1_ref, ah_ref, at_ref, deg_ref, w_ref,
                   c2_ref, gh2_ref, gt2_ref):
    deg = deg_ref[...]
    dh = _dinv(deg[:, 0])[:, None]
    dt = _dinv(deg[:, 1])[:, None]
    ah = jnp.concatenate([ah_ref[0], ah_ref[1]], axis=1)
    at = jnp.concatenate([at_ref[0], at_ref[1]], axis=1)
    h = c1_ref[...] + BETA * dh * ah + (1.0 - BETA) * dt * at
    h = jnp.maximum(h, 0.0)
    t2 = jnp.dot(h, w_ref[...], preferred_element_type=jnp.float32,
                 precision=lax.Precision.DEFAULT)
    c2_ref[...] = t2[:, :D_OUT]
    gh2_ref[...] = t2[:, D_OUT:2 * D_OUT] * dh
    gt2_ref[...] = t2[:, 2 * D_OUT:] * dt


def _combine2_body(c2_ref, ph_ref, pt_ref, deg_ref, probs_ref, logits_ref):
    deg = deg_ref[...]
    dh = _dinv(deg[:, 0])[:, None]
    dt = _dinv(deg[:, 1])[:, None]
    f = (c2_ref[...]
         + BETA * dh * (ph_ref[0] + ph_ref[1])
         + (1.0 - BETA) * dt * (pt_ref[0] + pt_ref[1]))
    m = jnp.max(f, axis=1, keepdims=True)
    e = jnp.exp(f - m)
    s = jnp.sum(e, axis=1, keepdims=True)
    probs_ref[...] = e / s
    logits_ref[...] = (f - m) - jnp.log(s)


def _row_spec(w):
    return pl.BlockSpec((BLK, w), lambda i: (i, 0))


def _pair_spec(w):
    return pl.BlockSpec((2, BLK, w), lambda i: (0, i, 0))


_deg_spec = pl.BlockSpec((BLK, 2), lambda i: (i, 0))


def _full_spec(h, w):
    return pl.BlockSpec((h, w), lambda i: (0, 0))


def _pad_idx(ei):
    npad = EP - E
    padr = (jnp.arange(npad, dtype=jnp.int32) * 97) % N
    padc = N + jnp.arange(npad, dtype=jnp.int32) % (NA - N)
    rows = jnp.concatenate([ei[0], padr]).reshape(NCHUNK, CH)
    cols = jnp.concatenate([ei[1], padc]).reshape(NCHUNK, CH)
    return rows, cols


def kernel(x, homo_edge_index, hetero_edge_index,
           W_center1, W_homo1, W_hetero1, W_center2, W_homo2, W_hetero2):
    rh, ch = _pad_idx(homo_edge_index)
    rt, ct = _pad_idx(hetero_edge_index)
    rh2, rt2 = rh * 2, rt * 2

    deg = _deg_kernel(ch, ct).T

    w1 = jnp.concatenate([W_center1, W_homo1, W_hetero1], axis=1)
    c1, gh1, gt1 = pl.pallas_call(
        _prep1_body,
        grid=(GRID,),
        in_specs=[_row_spec(D_IN), _full_spec(D_IN, 3 * D_HID), _deg_spec],
        out_specs=[_row_spec(D_HID)] * 3,
        out_shape=[jax.ShapeDtypeStruct((N, D_HID), jnp.float32)] * 3,
    )(x, w1, deg)

    a_h1, a_t1 = _conv_l1(gh1.reshape(2 * N, 128), gt1.reshape(2 * N, 128),
                          rh2, rh2 + 1, ch, rt2, rt2 + 1, ct)

    w2 = jnp.concatenate([W_center2, W_homo2, W_hetero2], axis=1)
    c2, gh2, gt2 = pl.pallas_call(
        _combine1_body,
        grid=(GRID,),
        in_specs=[_row_spec(D_HID), _pair_spec(128), _pair_spec(128),
                  _deg_spec, _full_spec(D_HID, 3 * D_OUT)],
        out_specs=[_row_spec(D_OUT)] * 3,
        out_shape=[jax.ShapeDtypeStruct((N, D_OUT), jnp.float32)] * 3,
    )(c1, a_h1, a_t1, deg, w2)

    p_h2, p_t2 = _conv_l2(gh2, gt2, rh, rh, ch, rt, rt, ct)

    probs, logits = pl.pallas_call(
        _combine2_body,
        grid=(GRID,),
        in_specs=[_row_spec(D_OUT), _pair_spec(D_OUT), _pair_spec(D_OUT),
                  _deg_spec],
        out_specs=[_row_spec(D_OUT)] * 2,
        out_shape=[jax.ShapeDtypeStruct((N, D_OUT), jnp.float32)] * 2,
    )(c2, p_h2, p_t2, deg)

    return (probs, logits)


# speedup vs baseline: 14.6503x; 1.0468x over previous
import functools

import jax
import jax.numpy as jnp
from jax import lax
from jax.experimental import pallas as pl
from jax.experimental.pallas import tpu as pltpu
from jax.experimental.pallas import tpu_sc as plsc

N = 10000
E = 160000
D_IN = 256
D_HID = 256
D_OUT = 128
BETA = 0.5

NC = 2
NS = 16
L = 16

CH = 128
NCHUNK = 1280
EP = NCHUNK * CH
NA = 10240
APT = NA // NS
WB = N // NS
NW = 40
BLK = 2000
GRID = N // BLK

_mesh = plsc.VectorSubcoreMesh(core_axis_name="c", subcore_axis_name="s")


def _fill(ref, n, vec):
    def body(i, _):
        ref[pl.ds(i * L, L)] = vec
        return 0
    lax.fori_loop(0, n // L, body, 0)


@functools.partial(
    pl.kernel,
    out_type=jax.ShapeDtypeStruct((NC, NA), jnp.float32),
    mesh=_mesh,
    scratch_types=[
        pltpu.VMEM((NCHUNK // NS, CH), jnp.int32),
        pltpu.VMEM((CH,), jnp.float32),
        pltpu.VMEM((APT,), jnp.float32),
        pltpu.VMEM_SHARED((NA,), jnp.float32),
    ],
)
def _deg_kernel(colh_hbm, colt_hbm, deg_out, idx_v, ones_v, zeros_v, acc):
    c = lax.axis_index("c")
    t = lax.axis_index("s")
    cpt = NCHUNK // NS
    _fill(ones_v, CH, jnp.ones((L,), jnp.float32))
    _fill(zeros_v, APT, jnp.zeros((L,), jnp.float32))
    pltpu.sync_copy(zeros_v, acc.at[pl.ds(t * APT, APT)])

    @pl.when(c == 0)
    def _():
        pltpu.sync_copy(colh_hbm.at[pl.ds(t * cpt, cpt)], idx_v)

    @pl.when(c == 1)
    def _():
        pltpu.sync_copy(colt_hbm.at[pl.ds(t * cpt, cpt)], idx_v)

    plsc.subcore_barrier()

    def body(j, _):
        pltpu.sync_copy(ones_v, acc.at[idx_v.at[j]], add=True)
        return 0
    lax.fori_loop(0, cpt, body, 0)

    plsc.subcore_barrier()
    pltpu.sync_copy(acc.at[pl.ds(t * APT, APT)],
                    deg_out.at[c, pl.ds(t * APT, APT)])


def _make_conv(split_features):
    cpt = NCHUNK // NS if split_features else NCHUNK // (NC * NS)

    @functools.partial(
        pl.kernel,
        out_type=[jax.ShapeDtypeStruct((NC, NA, 128), jnp.float32)] * 2,
        mesh=_mesh,
        scratch_types=[
            pltpu.VMEM((NW, CH), jnp.int32),
            pltpu.VMEM((NW, CH), jnp.int32),
            pltpu.VMEM((CH, 128), jnp.float32),
            pltpu.VMEM((CH, 128), jnp.float32),
            pltpu.VMEM_SHARED((NA, 128), jnp.float32),
            pltpu.SemaphoreType.DMA,
            pltpu.SemaphoreType.DMA,
        ],
    )
    def conv(gh_hbm, gt_hbm, rowh0, rowh1, colh, rowt0, rowt1, colt,
             outh, outt, idxr, idxc, bufa, bufb, acc, sema, semb):
        c = lax.axis_index("c")
        t = lax.axis_index("s")

        base = t * cpt if split_features else c * (NCHUNK // NC) + t * cpt

        def one_conv(g_hbm, row0_hbm, row1_hbm, col_hbm, out_hbm):
            def zb(i, _):
                bufa[i // 8, pl.ds((i % 8) * L, L)] = jnp.zeros((L,),
                                                                jnp.float32)
                return 0
            lax.fori_loop(0, CH * 8, zb, 0)
            for z in range(APT // CH):
                pltpu.sync_copy(bufa, acc.at[pl.ds(t * APT + z * CH, CH)])
            plsc.subcore_barrier()

            for w in range(cpt // NW):
                wbase = base + w * NW

                @pl.when(c == 0)
                def _():
                    pltpu.sync_copy(row0_hbm.at[pl.ds(wbase, NW)], idxr)

                @pl.when(c == 1)
                def _():
                    pltpu.sync_copy(row1_hbm.at[pl.ds(wbase, NW)], idxr)

                pltpu.sync_copy(col_hbm.at[pl.ds(wbase, NW)], idxc)

                pltpu.async_copy(g_hbm.at[idxr.at[0]], bufa, sema)

                def step(jj, _):
                    c0 = 2 * jj
                    c1 = 2 * jj + 1
                    pltpu.make_async_copy(g_hbm.at[idxr.at[0]], bufa,
                                          sema).wait()
                    pltpu.async_copy(g_hbm.at[idxr.at[c1]], bufb, semb)
                    pltpu.sync_copy(bufa, acc.at[idxc.at[c0]], add=True)
                    pltpu.make_async_copy(g_hbm.at[idxr.at[0]], bufb,
                                          semb).wait()
                    nxt = jnp.minimum(c0 + 2, NW - 1)
                    pltpu.async_copy(g_hbm.at[idxr.at[nxt]], bufa, sema)
                    pltpu.sync_copy(bufb, acc.at[idxc.at[c1]], add=True)
                    return 0
                lax.fori_loop(0, NW // 2, step, 0)
                pltpu.make_async_copy(g_hbm.at[idxr.at[0]], bufa, sema).wait()

            plsc.subcore_barrier()
            pltpu.sync_copy(acc.at[pl.ds(t * APT, APT)],
                            out_hbm.at[c, pl.ds(t * APT, APT)])
            plsc.subcore_barrier()

        one_conv(gh_hbm, rowh0, rowh1, colh, outh)
        one_conv(gt_hbm, rowt0, rowt1, colt, outt)

    return conv


_conv_l1 = _make_conv(True)
_conv_l2 = _make_conv(False)


def _dinv(d):
    return jnp.where(d > 0.0, lax.rsqrt(jnp.where(d > 0.0, d, 1.0)), 0.0)


def _prep1_body(x_ref, w_ref, deg_ref, c1_ref, gh_ref, gt_ref):
    t = jnp.dot(x_ref[...], w_ref[...], preferred_element_type=jnp.float32,
                precision=lax.Precision.DEFAULT)
    deg = deg_ref[...]
    dh = _dinv(deg[:, 0])[:, None]
    dt = _dinv(deg[:, 1])[:, None]
    c1_ref[...] = t[:, :D_HID]
    gh_ref[...] = (t[:, D_HID:2 * D_HID] * dh).reshape(2 * BLK, 128)
    gt_ref[...] = (t[:, 2 * D_HID:] * dt).reshape(2 * BLK, 128)


def _combine1_body(c1_ref, ah_ref, at_ref, deg_ref, w_ref,
                   c2_ref, gh2_ref, gt2_ref):
    deg = deg_ref[...]
    dh = _dinv(deg[:, 0])[:, None]
    dt = _dinv(deg[:, 1])[:, None]
    ah = jnp.concatenate([ah_ref[0], ah_ref[1]], axis=1)
    at = jnp.concatenate([at_ref[0], at_ref[1]], axis=1)
    h = c1_ref[...] + BETA * dh * ah + (1.0 - BETA) * dt * at
    h = jnp.maximum(h, 0.0)
    t2 = jnp.dot(h, w_ref[...], preferred_element_type=jnp.float32,
                 precision=lax.Precision.DEFAULT)
    c2_ref[...] = t2[:, :D_OUT]
    gh2_ref[...] = t2[:, D_OUT:2 * D_OUT] * dh
    gt2_ref[...] = t2[:, 2 * D_OUT:] * dt


def _combine2_body(c2_ref, ph_ref, pt_ref, deg_ref, probs_ref, logits_ref):
    deg = deg_ref[...]
    dh = _dinv(deg[:, 0])[:, None]
    dt = _dinv(deg[:, 1])[:, None]
    f = (c2_ref[...]
         + BETA * dh * (ph_ref[0] + ph_ref[1])
         + (1.0 - BETA) * dt * (pt_ref[0] + pt_ref[1]))
    m = jnp.max(f, axis=1, keepdims=True)
    e = jnp.exp(f - m)
    s = jnp.sum(e, axis=1, keepdims=True)
    probs_ref[...] = e / s
    logits_ref[...] = (f - m) - jnp.log(s)


def _row_spec(w):
    return pl.BlockSpec((BLK, w), lambda i: (i, 0))


def _pair_spec(w):
    return pl.BlockSpec((2, BLK, w), lambda i: (0, i, 0))


_deg_spec = pl.BlockSpec((BLK, 2), lambda i: (i, 0))


def _full_spec(h, w):
    return pl.BlockSpec((h, w), lambda i: (0, 0))


def _pad_idx(ei):
    npad = EP - E
    padr = (jnp.arange(npad, dtype=jnp.int32) * 97) % N
    padc = N + jnp.arange(npad, dtype=jnp.int32) % (NA - N)
    rows = jnp.concatenate([ei[0], padr]).reshape(NCHUNK, CH)
    cols = jnp.concatenate([ei[1], padc]).reshape(NCHUNK, CH)
    return rows, cols


def kernel(x, homo_edge_index, hetero_edge_index,
           W_center1, W_homo1, W_hetero1, W_center2, W_homo2, W_hetero2):
    rh, ch = _pad_idx(homo_edge_index)
    rt, ct = _pad_idx(hetero_edge_index)
    rh2, rt2 = rh * 2, rt * 2

    deg = _deg_kernel(ch, ct).T

    w1 = jnp.concatenate([W_center1, W_homo1, W_hetero1], axis=1)
    c1, gh1, gt1 = pl.pallas_call(
        _prep1_body,
        grid=(GRID,),
        in_specs=[_row_spec(D_IN), _full_spec(D_IN, 3 * D_HID), _deg_spec],
        out_specs=[_row_spec(D_HID),
                   pl.BlockSpec((2 * BLK, 128), lambda i: (i, 0)),
                   pl.BlockSpec((2 * BLK, 128), lambda i: (i, 0))],
        out_shape=[jax.ShapeDtypeStruct((N, D_HID), jnp.float32),
                   jax.ShapeDtypeStruct((2 * N, 128), jnp.float32),
                   jax.ShapeDtypeStruct((2 * N, 128), jnp.float32)],
    )(x, w1, deg)

    a_h1, a_t1 = _conv_l1(gh1, gt1,
                          rh2, rh2 + 1, ch, rt2, rt2 + 1, ct)

    w2 = jnp.concatenate([W_center2, W_homo2, W_hetero2], axis=1)
    c2, gh2, gt2 = pl.pallas_call(
        _combine1_body,
        grid=(GRID,),
        in_specs=[_row_spec(D_HID), _pair_spec(128), _pair_spec(128),
                  _deg_spec, _full_spec(D_HID, 3 * D_OUT)],
        out_specs=[_row_spec(D_OUT)] * 3,
        out_shape=[jax.ShapeDtypeStruct((N, D_OUT), jnp.float32)] * 3,
    )(c1, a_h1, a_t1, deg, w2)

    p_h2, p_t2 = _conv_l2(gh2, gt2, rh, rh, ch, rt, rt, ct)

    probs, logits = pl.pallas_call(
        _combine2_body,
        grid=(GRID,),
        in_specs=[_row_spec(D_OUT), _pair_spec(D_OUT), _pair_spec(D_OUT),
                  _deg_spec],
        out_specs=[_row_spec(D_OUT)] * 2,
        out_shape=[jax.ShapeDtypeStruct((N, D_OUT), jnp.float32)] * 2,
    )(c2, p_h2, p_t2, deg)

    return (probs, logits)
